# baseline (speedup 1.0000x reference)
import jax
import jax.numpy as jnp
from jax import lax
from jax.experimental import pallas as pl
from jax.experimental.pallas import tpu as pltpu
from jax.experimental.pallas import tpu_sc as plsc

_BATCH = 16384
_N_FIELDS = 26
_VOCAB = 100000
_EMB_DIM = 16
_N_CONT = 10
_OUT_W = _N_FIELDS * _EMB_DIM + _N_CONT

_NC = 2
_NS = 16
_LANES = 16
_NW = _NC * _NS
_BPW = _BATCH // _NW
_GRP = _BPW // _LANES


def _body(tab_hbm, xcatt_hbm, xcontt_hbm, cmin_hbm, cmax_hbm, outt_hbm,
          xc_v, idx0_v, idx1_v, fc0_v, fc1_v, fct_v,
          cin_v, cm_v, cx_v, sem0, sem1):
    wid = lax.axis_index("s") * _NC + lax.axis_index("c")
    base = wid * _BPW
    iota = lax.iota(jnp.int32, _LANES)

    pltpu.sync_copy(xcatt_hbm.at[:, pl.ds(base, _BPW)], xc_v)

    idxv = (idx0_v, idx1_v)
    fcv = (fc0_v, fc1_v)
    sems = (sem0, sem1)
    desc = [None, None]

    def build_idx(f):
        idxr = idxv[f & 1]
        fvec = jnp.full((_LANES,), f, jnp.int32)

        @pl.loop(0, _GRP)
        def _(g):
            rvec = g * _LANES + iota
            col = plsc.load_gather(xc_v, [fvec, rvec])
            idxr[pl.ds(g * _LANES, _LANES)] = col + f * _VOCAB

    evecs = [jnp.full((_LANES,), e, jnp.int32) for e in range(_EMB_DIM)]

    def emit_field(f):
        fcr = fcv[f & 1]

        @pl.loop(0, _GRP)
        def _(g):
            jvec = g * _LANES + iota
            s = pl.ds(g * _LANES, _LANES)
            for e in range(_EMB_DIM):
                fct_v[e, s] = plsc.load_gather(fcr, [jvec, evecs[e]])

        pltpu.sync_copy(
            fct_v,
            outt_hbm.at[pl.ds(f * _EMB_DIM, _EMB_DIM), pl.ds(base, _BPW)])

    def cont_path():
        pltpu.sync_copy(xcontt_hbm.at[:, pl.ds(base, _BPW)], cin_v)
        pltpu.sync_copy(cmin_hbm, cm_v.at[pl.ds(0, _N_CONT)])
        pltpu.sync_copy(cmax_hbm, cx_v.at[pl.ds(0, _N_CONT)])
        mnv = cm_v[...]
        mxv = cx_v[...]
        for c in range(_N_CONT):
            mn = mnv[c]
            den = mxv[c] - mn

            @pl.loop(0, _GRP)
            def _(g, c=c, mn=mn, den=den):
                s = pl.ds(g * _LANES, _LANES)
                cin_v[c, s] = (cin_v[c, s] - mn) / den

        pltpu.sync_copy(
            cin_v,
            outt_hbm.at[pl.ds(_N_FIELDS * _EMB_DIM, _N_CONT),
                        pl.ds(base, _BPW)])

    for f in range(_N_FIELDS):
        cur = f & 1
        build_idx(f)
        desc[cur] = pltpu.async_copy(tab_hbm.at[idxv[cur]], fcv[cur],
                                     sems[cur])
        if f == 0:
            cont_path()
        if f >= 1:
            desc[1 - cur].wait()
            emit_field(f - 1)
    desc[1].wait()
    emit_field(_N_FIELDS - 1)


_mesh = plsc.VectorSubcoreMesh(core_axis_name="c", subcore_axis_name="s")

_sc_call = pl.kernel(
    _body,
    out_type=jax.ShapeDtypeStruct((_OUT_W, _BATCH), jnp.float32),
    mesh=_mesh,
    scratch_types=[
        pltpu.VMEM((_N_FIELDS, _BPW), jnp.int32),
        pltpu.VMEM((_BPW,), jnp.int32),
        pltpu.VMEM((_BPW,), jnp.int32),
        pltpu.VMEM((_BPW, _EMB_DIM), jnp.float32),
        pltpu.VMEM((_BPW, _EMB_DIM), jnp.float32),
        pltpu.VMEM((_EMB_DIM, _BPW), jnp.float32),
        pltpu.VMEM((_N_CONT, _BPW), jnp.float32),
        pltpu.VMEM((_LANES,), jnp.float32),
        pltpu.VMEM((_LANES,), jnp.float32),
        pltpu.SemaphoreType.DMA,
        pltpu.SemaphoreType.DMA,
    ],
    compiler_params=pltpu.CompilerParams(
        use_tc_tiling_on_sc=False, needs_layout_passes=False),
)


@jax.jit
def kernel(x_cat, x_cont, tables, cont_min, cont_max):
    xcatt = x_cat.astype(jnp.int32).T
    xcontt = x_cont.T
    tab_flat = tables.reshape(_N_FIELDS * _VOCAB, _EMB_DIM)
    outt = _sc_call(tab_flat, xcatt, xcontt, cont_min, cont_max)
    return outt.T
